# Initial kernel scaffold; baseline (speedup 1.0000x reference)
import jax
import jax.numpy as jnp
from jax.experimental import pallas as pl


def kernel(data, length, embed_w, lin_w, lin_b):
    raise NotImplementedError("write your pallas kernel here")


# speedup vs baseline: 16.1281x; 16.1281x over previous
import functools

import jax
import jax.numpy as jnp
from jax import lax
from jax.experimental import pallas as pl
from jax.experimental.pallas import tpu as pltpu
from jax.experimental.pallas import tpu_sc as plsc

VOCAB = 1_000_000
EMB = 32
BATCH = 16384
L = 200
NCLASS = 20

NC = 2
NS = 16
NW = NC * NS
RW = BATCH // NW
SUP = 64
NBUF = 4
H0 = 96
H1 = L - H0


def _sc_pool(data_i32, embed_w):
    mesh = plsc.VectorSubcoreMesh(
        core_axis_name="c", subcore_axis_name="s",
        num_cores=NC, num_subcores=NS)

    @functools.partial(
        pl.kernel,
        out_type=jax.ShapeDtypeStruct((BATCH, EMB), jnp.float32),
        mesh=mesh,
        compiler_params=pltpu.CompilerParams(use_tc_tiling_on_sc=False),
        scratch_types=[
            pltpu.VMEM((SUP, L), jnp.int32),
            pltpu.VMEM((NBUF, L, EMB), jnp.float32),
            pltpu.VMEM((RW, EMB), jnp.float32),
        ] + [pltpu.SemaphoreType.DMA] * NBUF,
    )
    def k(data_hbm, table_hbm, out_hbm, idx_v, rows_v, out_v, *sems):
        wid = lax.axis_index("s") * NC + lax.axis_index("c")
        base = wid * RW

        def fire(rloc, b):
            pltpu.async_copy(
                table_hbm.at[idx_v.at[rloc, pl.ds(0, H0)]],
                rows_v.at[b, pl.ds(0, H0), :], sems[b])
            pltpu.async_copy(
                table_hbm.at[idx_v.at[rloc, pl.ds(H0, H1)]],
                rows_v.at[b, pl.ds(H0, H1), :], sems[b])

        def drain(b):
            pltpu.make_async_copy(
                table_hbm.at[idx_v.at[0, pl.ds(0, H0)]],
                rows_v.at[b, pl.ds(0, H0), :], sems[b]).wait()
            pltpu.make_async_copy(
                table_hbm.at[idx_v.at[0, pl.ds(H0, H1)]],
                rows_v.at[b, pl.ds(H0, H1), :], sems[b]).wait()

        def accum(b, rout):
            z = jnp.zeros((16,), jnp.float32)

            def body(j, carry):
                a00, a01, a10, a11 = carry
                r2 = j * 2
                a00 = a00 + rows_v[b, r2, pl.ds(0, 16)]
                a01 = a01 + rows_v[b, r2, pl.ds(16, 16)]
                a10 = a10 + rows_v[b, r2 + 1, pl.ds(0, 16)]
                a11 = a11 + rows_v[b, r2 + 1, pl.ds(16, 16)]
                return a00, a01, a10, a11

            a00, a01, a10, a11 = lax.fori_loop(
                0, L // 2, body, (z, z, z, z), unroll=4)
            out_v[rout, pl.ds(0, 16)] = a00 + a10
            out_v[rout, pl.ds(16, 16)] = a01 + a11

        @pl.loop(0, RW, step=SUP)
        def super_block(row0):
            pltpu.sync_copy(data_hbm.at[pl.ds(base + row0, SUP)], idx_v)
            for b in range(NBUF):
                fire(b, b)

            @pl.loop(0, SUP, step=NBUF)
            def inner(g):
                for b in range(NBUF):
                    rl = g + b
                    drain(b)
                    accum(b, row0 + rl)

                    @pl.when(rl + NBUF < SUP)
                    def _():
                        fire(rl + NBUF, b)

        pltpu.sync_copy(out_v, out_hbm.at[pl.ds(base, RW)])

    return k(data_i32, embed_w)


def _tc_tail(pooled, length_col, wt, bias):
    blk = 2048
    grid = BATCH // blk

    def body(p_ref, len_ref, w_ref, b_ref, o_ref):
        inv = 1.0 / len_ref[...].astype(jnp.float32)
        x = p_ref[...] * inv
        o_ref[...] = (jnp.dot(x, w_ref[...],
                              preferred_element_type=jnp.float32)
                      + b_ref[0, :][None, :])

    return pl.pallas_call(
        body,
        grid=(grid,),
        in_specs=[
            pl.BlockSpec((blk, EMB), lambda i: (i, 0)),
            pl.BlockSpec((blk, 1), lambda i: (i, 0)),
            pl.BlockSpec((EMB, NCLASS), lambda i: (0, 0)),
            pl.BlockSpec((8, NCLASS), lambda i: (0, 0)),
        ],
        out_specs=pl.BlockSpec((blk, NCLASS), lambda i: (i, 0)),
        out_shape=jax.ShapeDtypeStruct((BATCH, NCLASS), jnp.float32),
    )(pooled, length_col, wt, bias)


def kernel(data, length, embed_w, lin_w, lin_b):
    data_i32 = data.astype(jnp.int32)
    pooled = _sc_pool(data_i32, embed_w)
    length_col = length.astype(jnp.int32).reshape(BATCH, 1)
    wt = lin_w.T
    bias = jnp.tile(lin_b[None, :], (8, 1))
    return _tc_tail(pooled, length_col, wt, bias)


# speedup vs baseline: 36.1951x; 2.2442x over previous
import functools

import jax
import jax.numpy as jnp
from jax import lax
from jax.experimental import pallas as pl
from jax.experimental.pallas import tpu as pltpu
from jax.experimental.pallas import tpu_sc as plsc

VOCAB = 1_000_000
EMB = 32
BATCH = 16384
L = 200
NCLASS = 20

NC = 2
NS = 16
NW = NC * NS
RW = BATCH // NW
SUP = 64
NBUF = 8
H0 = 96
H1 = L - H0
W16 = EMB // 2

DET_C = 8192
SPL = DET_C // 8
DET_GRID = (VOCAB + DET_C - 1) // DET_C
TAIL_ROWS = VOCAB - (DET_GRID - 1) * DET_C
NROW128 = (DET_GRID - 1) * SPL + TAIL_ROWS
NTAB = NROW128 * 8


def _tc_detile(embed_wt):
    def body(x_ref, o_ref):
        x = x_ref[...]
        xi = pltpu.bitcast(x.astype(jnp.bfloat16), jnp.int32)
        xx = jnp.concatenate(
            [xi[:, a * SPL:(a + 1) * SPL] for a in range(8)],
            axis=0)
        o_ref[...] = xx.T

    return pl.pallas_call(
        body,
        grid=(DET_GRID,),
        in_specs=[pl.BlockSpec((EMB, DET_C), lambda i: (0, i))],
        out_specs=pl.BlockSpec((SPL, 128), lambda i: (i, 0)),
        out_shape=jax.ShapeDtypeStruct((NROW128, 128), jnp.int32),
    )(embed_wt)


def _sc_pool(fidx, table):
    mesh = plsc.VectorSubcoreMesh(
        core_axis_name="c", subcore_axis_name="s",
        num_cores=NC, num_subcores=NS)

    @functools.partial(
        pl.kernel,
        out_type=jax.ShapeDtypeStruct((BATCH, EMB), jnp.float32),
        mesh=mesh,
        compiler_params=pltpu.CompilerParams(use_tc_tiling_on_sc=False),
        scratch_types=[
            pltpu.VMEM((SUP, L), jnp.int32),
            pltpu.VMEM((NBUF, L, W16), jnp.int32),
            pltpu.VMEM((RW, EMB), jnp.float32),
        ] + [pltpu.SemaphoreType.DMA] * NBUF,
    )
    def k(data_hbm, table_hbm, out_hbm, idx_v, rows_v, out_v, *sems):
        wid = lax.axis_index("s") * NC + lax.axis_index("c")
        base = wid * RW

        def fire(rloc, b):
            pltpu.async_copy(
                table_hbm.at[idx_v.at[rloc, pl.ds(0, H0)]],
                rows_v.at[b, pl.ds(0, H0), :], sems[b])
            pltpu.async_copy(
                table_hbm.at[idx_v.at[rloc, pl.ds(H0, H1)]],
                rows_v.at[b, pl.ds(H0, H1), :], sems[b])

        def drain(b):
            pltpu.make_async_copy(
                table_hbm.at[idx_v.at[0, pl.ds(0, H0)]],
                rows_v.at[b, pl.ds(0, H0), :], sems[b]).wait()
            pltpu.make_async_copy(
                table_hbm.at[idx_v.at[0, pl.ds(H0, H1)]],
                rows_v.at[b, pl.ds(H0, H1), :], sems[b]).wait()

        def accum(b, rout):
            z = jnp.zeros((16,), jnp.float32)

            hi = jnp.int32(-65536)

            def body(j, carry):
                ae0, ao0, ae1, ao1 = carry
                r2 = j * 2
                w0 = rows_v[b, r2, :]
                w1 = rows_v[b, r2 + 1, :]
                e0 = lax.bitcast_convert_type(w0 << 16, jnp.float32)
                o0 = lax.bitcast_convert_type(w0 & hi, jnp.float32)
                e1 = lax.bitcast_convert_type(w1 << 16, jnp.float32)
                o1 = lax.bitcast_convert_type(w1 & hi, jnp.float32)
                return ae0 + e0, ao0 + o0, ae1 + e1, ao1 + o1

            ae0, ao0, ae1, ao1 = lax.fori_loop(
                0, L // 2, body, (z, z, z, z), unroll=4)
            out_v[rout, pl.ds(0, 16)] = ae0 + ae1
            out_v[rout, pl.ds(16, 16)] = ao0 + ao1

        @pl.loop(0, RW, step=SUP)
        def super_block(row0):
            pltpu.sync_copy(data_hbm.at[pl.ds(base + row0, SUP)], idx_v)
            for b in range(NBUF):
                fire(b, b)

            @pl.loop(0, SUP, step=NBUF)
            def inner(g):
                for b in range(NBUF):
                    rl = g + b
                    drain(b)
                    accum(b, row0 + rl)

                    @pl.when(rl + NBUF < SUP)
                    def _():
                        fire(rl + NBUF, b)

        pltpu.sync_copy(out_v, out_hbm.at[pl.ds(base, RW)])

    return k(fidx, table)


def _tc_tail(pooled, length_col, wt, bias):
    blk = 2048
    grid = BATCH // blk

    def body(p_ref, len_ref, w_ref, b_ref, o_ref):
        inv = 1.0 / len_ref[...].astype(jnp.float32)
        x = p_ref[...] * inv
        o_ref[...] = (jnp.dot(x, w_ref[...],
                              preferred_element_type=jnp.float32)
                      + b_ref[0, :][None, :])

    return pl.pallas_call(
        body,
        grid=(grid,),
        in_specs=[
            pl.BlockSpec((blk, EMB), lambda i: (i, 0)),
            pl.BlockSpec((blk, 1), lambda i: (i, 0)),
            pl.BlockSpec((EMB, NCLASS), lambda i: (0, 0)),
            pl.BlockSpec((8, NCLASS), lambda i: (0, 0)),
        ],
        out_specs=pl.BlockSpec((blk, NCLASS), lambda i: (i, 0)),
        out_shape=jax.ShapeDtypeStruct((BATCH, NCLASS), jnp.float32),
    )(pooled, length_col, wt, bias)


def kernel(data, length, embed_w, lin_w, lin_b):
    d = data.astype(jnp.int32)
    fidx = ((d >> 13) << 13) + ((d & 1023) << 3) + ((d >> 10) & 7)
    table = jax.lax.optimization_barrier(
        _tc_detile(embed_w.T)).reshape(NTAB, W16)
    pooled = _sc_pool(fidx, table)
    length_col = length.astype(jnp.int32).reshape(BATCH, 1)
    perm = jnp.arange(EMB).reshape(EMB // 2, 2).T.reshape(EMB)
    wt = lin_w.T[perm, :]
    bias = jnp.tile(lin_b[None, :], (8, 1))
    return _tc_tail(pooled, length_col, wt, bias)


# speedup vs baseline: 36.4217x; 1.0063x over previous
import functools

import jax
import jax.numpy as jnp
from jax import lax
from jax.experimental import pallas as pl
from jax.experimental.pallas import tpu as pltpu
from jax.experimental.pallas import tpu_sc as plsc

VOCAB = 1_000_000
EMB = 32
BATCH = 16384
L = 200
NCLASS = 20

NC = 2
NS = 16
NW = NC * NS
RW = BATCH // NW
SUP = 64
NBUF = 8
H0 = 96
H1 = L - H0
W16 = EMB // 2

DET_C = 8192
SPL = DET_C // 8
DET_GRID = (VOCAB + DET_C - 1) // DET_C
TAIL_ROWS = VOCAB - (DET_GRID - 1) * DET_C
NROW128 = (DET_GRID - 1) * SPL + TAIL_ROWS
NTAB = NROW128 * 8


def _tc_detile(embed_wt):
    def body(x_ref, o_ref):
        x = x_ref[...]
        xi = pltpu.bitcast(x.astype(jnp.bfloat16), jnp.int32)
        xx = jnp.concatenate(
            [xi[:, a * SPL:(a + 1) * SPL] for a in range(8)],
            axis=0)
        o_ref[...] = xx.T

    return pl.pallas_call(
        body,
        grid=(DET_GRID,),
        in_specs=[pl.BlockSpec((EMB, DET_C), lambda i: (0, i))],
        out_specs=pl.BlockSpec((SPL, 128), lambda i: (i, 0)),
        out_shape=jax.ShapeDtypeStruct((NROW128, 128), jnp.int32),
    )(embed_wt)


def _sc_pool(fidx, table):
    mesh = plsc.VectorSubcoreMesh(
        core_axis_name="c", subcore_axis_name="s",
        num_cores=NC, num_subcores=NS)

    @functools.partial(
        pl.kernel,
        out_type=jax.ShapeDtypeStruct((BATCH, EMB), jnp.float32),
        mesh=mesh,
        compiler_params=pltpu.CompilerParams(use_tc_tiling_on_sc=False),
        scratch_types=[
            pltpu.VMEM((SUP, L), jnp.int32),
            pltpu.VMEM((NBUF, L, W16), jnp.int32),
            pltpu.VMEM((RW, EMB), jnp.float32),
        ] + [pltpu.SemaphoreType.DMA] * NBUF,
    )
    def k(data_hbm, table_hbm, out_hbm, idx_v, rows_v, out_v, *sems):
        wid = lax.axis_index("s") * NC + lax.axis_index("c")
        base = wid * RW

        def fire(rloc, b):
            pltpu.async_copy(
                table_hbm.at[idx_v.at[rloc, pl.ds(0, H0)]],
                rows_v.at[b, pl.ds(0, H0), :], sems[b])
            pltpu.async_copy(
                table_hbm.at[idx_v.at[rloc, pl.ds(H0, H1)]],
                rows_v.at[b, pl.ds(H0, H1), :], sems[b])

        def drain(b):
            pltpu.make_async_copy(
                table_hbm.at[idx_v.at[0, pl.ds(0, H0)]],
                rows_v.at[b, pl.ds(0, H0), :], sems[b]).wait()
            pltpu.make_async_copy(
                table_hbm.at[idx_v.at[0, pl.ds(H0, H1)]],
                rows_v.at[b, pl.ds(H0, H1), :], sems[b]).wait()

        def accum(b, rout):
            z = jnp.zeros((16,), jnp.float32)

            def body(j, carry):
                ae0, ao0, ae1, ao1 = carry
                r2 = j * 2
                w0 = rows_v[b, r2, :]
                w1 = rows_v[b, r2 + 1, :]
                e0 = lax.bitcast_convert_type(w0 << 16, jnp.float32)
                o0 = lax.bitcast_convert_type(w0, jnp.float32)
                e1 = lax.bitcast_convert_type(w1 << 16, jnp.float32)
                o1 = lax.bitcast_convert_type(w1, jnp.float32)
                return ae0 + e0, ao0 + o0, ae1 + e1, ao1 + o1

            ae0, ao0, ae1, ao1 = lax.fori_loop(
                0, L // 2, body, (z, z, z, z), unroll=4)
            out_v[rout, pl.ds(0, 16)] = ae0 + ae1
            out_v[rout, pl.ds(16, 16)] = ao0 + ao1

        @pl.loop(0, RW, step=SUP)
        def super_block(row0):
            pltpu.sync_copy(data_hbm.at[pl.ds(base + row0, SUP)], idx_v)
            for b in range(NBUF):
                fire(b, b)

            @pl.loop(0, SUP, step=NBUF)
            def inner(g):
                for b in range(NBUF):
                    rl = g + b
                    drain(b)
                    accum(b, row0 + rl)

                    @pl.when(rl + NBUF < SUP)
                    def _():
                        fire(rl + NBUF, b)

        pltpu.sync_copy(out_v, out_hbm.at[pl.ds(base, RW)])

    return k(fidx, table)


def _tc_tail(pooled, length_col, wt, bias):
    blk = 2048
    grid = BATCH // blk

    def body(p_ref, len_ref, w_ref, b_ref, o_ref):
        inv = 1.0 / len_ref[...].astype(jnp.float32)
        x = p_ref[...] * inv
        o_ref[...] = (jnp.dot(x, w_ref[...],
                              preferred_element_type=jnp.float32)
                      + b_ref[0, :][None, :])

    return pl.pallas_call(
        body,
        grid=(grid,),
        in_specs=[
            pl.BlockSpec((blk, EMB), lambda i: (i, 0)),
            pl.BlockSpec((blk, 1), lambda i: (i, 0)),
            pl.BlockSpec((EMB, NCLASS), lambda i: (0, 0)),
            pl.BlockSpec((8, NCLASS), lambda i: (0, 0)),
        ],
        out_specs=pl.BlockSpec((blk, NCLASS), lambda i: (i, 0)),
        out_shape=jax.ShapeDtypeStruct((BATCH, NCLASS), jnp.float32),
    )(pooled, length_col, wt, bias)


def kernel(data, length, embed_w, lin_w, lin_b):
    d = data.astype(jnp.int32)
    fidx = ((d >> 13) << 13) + ((d & 1023) << 3) + ((d >> 10) & 7)
    table = jax.lax.optimization_barrier(
        _tc_detile(embed_w.T)).reshape(NTAB, W16)
    pooled = _sc_pool(fidx, table)
    length_col = length.astype(jnp.int32).reshape(BATCH, 1)
    perm = jnp.arange(EMB).reshape(EMB // 2, 2).T.reshape(EMB)
    wt = lin_w.T[perm, :]
    bias = jnp.tile(lin_b[None, :], (8, 1))
    return _tc_tail(pooled, length_col, wt, bias)


# speedup vs baseline: 37.1464x; 1.0199x over previous
import functools

import jax
import jax.numpy as jnp
from jax import lax
from jax.experimental import pallas as pl
from jax.experimental.pallas import tpu as pltpu
from jax.experimental.pallas import tpu_sc as plsc

VOCAB = 1_000_000
EMB = 32
BATCH = 16384
L = 200
NCLASS = 20

NC = 2
NS = 16
NW = NC * NS
RW = BATCH // NW
SUP = 64
NBUF = 8
H0 = 96
H1 = L - H0
W16 = EMB // 2

LPAD = 256
FI_BLK = 2048

DET_C = 8192
SPL = DET_C // 8
DET_GRID = (VOCAB + DET_C - 1) // DET_C
TAIL_ROWS = VOCAB - (DET_GRID - 1) * DET_C
NROW128 = (DET_GRID - 1) * SPL + TAIL_ROWS
NTAB = NROW128 * 8


def _tc_detile(embed_wt):
    def body(x_ref, o_ref):
        x = x_ref[...]
        xi = pltpu.bitcast(x.astype(jnp.bfloat16), jnp.int32)
        xx = jnp.concatenate(
            [xi[:, a * SPL:(a + 1) * SPL] for a in range(8)],
            axis=0)
        o_ref[...] = xx.T

    return pl.pallas_call(
        body,
        grid=(DET_GRID,),
        in_specs=[pl.BlockSpec((EMB, DET_C), lambda i: (0, i))],
        out_specs=pl.BlockSpec((SPL, 128), lambda i: (i, 0)),
        out_shape=jax.ShapeDtypeStruct((NROW128, 128), jnp.int32),
    )(embed_wt)


def _tc_fidx(d):
    def body(x_ref, o_ref):
        x = x_ref[...]
        f = ((x >> 13) << 13) + ((x & 1023) << 3) + ((x >> 10) & 7)
        o_ref[:, :L] = f

    return pl.pallas_call(
        body,
        grid=(BATCH // FI_BLK,),
        in_specs=[pl.BlockSpec((FI_BLK, L), lambda i: (i, 0))],
        out_specs=pl.BlockSpec((FI_BLK, LPAD), lambda i: (i, 0)),
        out_shape=jax.ShapeDtypeStruct((BATCH, LPAD), jnp.int32),
    )(d)


def _sc_pool(fidx, table):
    mesh = plsc.VectorSubcoreMesh(
        core_axis_name="c", subcore_axis_name="s",
        num_cores=NC, num_subcores=NS)

    @functools.partial(
        pl.kernel,
        out_type=jax.ShapeDtypeStruct((BATCH, EMB), jnp.float32),
        mesh=mesh,
        compiler_params=pltpu.CompilerParams(use_tc_tiling_on_sc=False),
        scratch_types=[
            pltpu.VMEM((SUP, LPAD), jnp.int32),
            pltpu.VMEM((NBUF, L, W16), jnp.int32),
            pltpu.VMEM((RW, EMB), jnp.float32),
        ] + [pltpu.SemaphoreType.DMA] * NBUF,
    )
    def k(data_hbm, table_hbm, out_hbm, idx_v, rows_v, out_v, *sems):
        wid = lax.axis_index("s") * NC + lax.axis_index("c")
        base = wid * RW

        def fire(rloc, b):
            pltpu.async_copy(
                table_hbm.at[idx_v.at[rloc, pl.ds(0, H0)]],
                rows_v.at[b, pl.ds(0, H0), :], sems[b])
            pltpu.async_copy(
                table_hbm.at[idx_v.at[rloc, pl.ds(H0, H1)]],
                rows_v.at[b, pl.ds(H0, H1), :], sems[b])

        def drain(b):
            pltpu.make_async_copy(
                table_hbm.at[idx_v.at[0, pl.ds(0, H0)]],
                rows_v.at[b, pl.ds(0, H0), :], sems[b]).wait()
            pltpu.make_async_copy(
                table_hbm.at[idx_v.at[0, pl.ds(H0, H1)]],
                rows_v.at[b, pl.ds(H0, H1), :], sems[b]).wait()

        def accum(b, rout):
            z = jnp.zeros((16,), jnp.float32)

            def body(j, carry):
                ae0, ao0, ae1, ao1 = carry
                r2 = j * 2
                w0 = rows_v[b, r2, :]
                w1 = rows_v[b, r2 + 1, :]
                e0 = lax.bitcast_convert_type(w0 << 16, jnp.float32)
                o0 = lax.bitcast_convert_type(w0, jnp.float32)
                e1 = lax.bitcast_convert_type(w1 << 16, jnp.float32)
                o1 = lax.bitcast_convert_type(w1, jnp.float32)
                return ae0 + e0, ao0 + o0, ae1 + e1, ao1 + o1

            ae0, ao0, ae1, ao1 = lax.fori_loop(
                0, L // 2, body, (z, z, z, z), unroll=4)
            out_v[rout, pl.ds(0, 16)] = ae0 + ae1
            out_v[rout, pl.ds(16, 16)] = ao0 + ao1

        @pl.loop(0, RW, step=SUP)
        def super_block(row0):
            pltpu.sync_copy(data_hbm.at[pl.ds(base + row0, SUP)], idx_v)
            for b in range(NBUF):
                fire(b, b)

            @pl.loop(0, SUP, step=NBUF)
            def inner(g):
                for b in range(NBUF):
                    rl = g + b
                    drain(b)
                    accum(b, row0 + rl)

                    @pl.when(rl + NBUF < SUP)
                    def _():
                        fire(rl + NBUF, b)

        pltpu.sync_copy(out_v, out_hbm.at[pl.ds(base, RW)])

    return k(fidx, table)


def _tc_tail(pooled, length_col, wt, bias):
    blk = 2048
    grid = BATCH // blk

    def body(p_ref, len_ref, w_ref, b_ref, o_ref):
        inv = 1.0 / len_ref[...].astype(jnp.float32)
        x = p_ref[...] * inv
        o_ref[...] = (jnp.dot(x, w_ref[...],
                              preferred_element_type=jnp.float32)
                      + b_ref[0, :][None, :])

    return pl.pallas_call(
        body,
        grid=(grid,),
        in_specs=[
            pl.BlockSpec((blk, EMB), lambda i: (i, 0)),
            pl.BlockSpec((blk, 1), lambda i: (i, 0)),
            pl.BlockSpec((EMB, NCLASS), lambda i: (0, 0)),
            pl.BlockSpec((8, NCLASS), lambda i: (0, 0)),
        ],
        out_specs=pl.BlockSpec((blk, NCLASS), lambda i: (i, 0)),
        out_shape=jax.ShapeDtypeStruct((BATCH, NCLASS), jnp.float32),
    )(pooled, length_col, wt, bias)


def kernel(data, length, embed_w, lin_w, lin_b):
    d = data.astype(jnp.int32)
    fidx = jax.lax.optimization_barrier(_tc_fidx(d))
    table = jax.lax.optimization_barrier(
        _tc_detile(embed_w.T)).reshape(NTAB, W16)
    pooled = _sc_pool(fidx, table)
    length_col = length.astype(jnp.int32).reshape(BATCH, 1)
    perm = jnp.arange(EMB).reshape(EMB // 2, 2).T.reshape(EMB)
    wt = lin_w.T[perm, :]
    bias = jnp.tile(lin_b[None, :], (8, 1))
    return _tc_tail(pooled, length_col, wt, bias)
